# Initial kernel scaffold; baseline (speedup 1.0000x reference)
import jax
import jax.numpy as jnp
from jax.experimental import pallas as pl


def kernel(x_track, x_artist, x_genre, ei_made_by, ei_rev_made_by, ei_tagged, ei_rev_tagged, ei_plays, ei_rev_plays, ei_co_genre, proj_W, proj_b, conv1_Wl, conv1_bl, conv1_Wr, conv2_Wl, conv2_bl, conv2_Wr, norm1_g, norm1_b, norm2_g, norm2_b, head_W1, head_b1, head_W2, head_b2):
    raise NotImplementedError("write your pallas kernel here")


# speedup vs baseline: 1.0752x; 1.0752x over previous
import jax
import jax.numpy as jnp
from jax.experimental import pallas as pl

_NTYPES = ['track', 'artist', 'genre']
_N_NODES = {'track': 100000, 'artist': 10000, 'genre': 1000}
_EDGE_TYPES = [('track', 'artist', 'ei_made_by'), ('artist', 'track', 'ei_rev_made_by'), ('track', 'genre', 'ei_tagged'), ('genre', 'track', 'ei_rev_tagged'), ('artist', 'genre', 'ei_plays'), ('genre', 'artist', 'ei_rev_plays'), ('track', 'track', 'ei_co_genre')]


def _layernorm(x, g, b):
    m = jnp.mean(x, axis=-1, keepdims=True)
    v = jnp.var(x, axis=-1, keepdims=True)
    return (x - m) / jnp.sqrt(v + 1e-5) * g + b


def _sage(x_src, x_dst, src, dst, Wl, bl, Wr):
    n_dst = x_dst.shape[0]
    msgs = jnp.take(x_src, src, axis=0)
    agg = jax.ops.segment_sum(msgs, dst, num_segments=n_dst, indices_are_sorted=True)
    cnt = jax.ops.segment_sum(jnp.ones((src.shape[0],), msgs.dtype), dst, num_segments=n_dst, indices_are_sorted=True)
    mean = agg / jnp.maximum(cnt, 1.0)[:, None]
    return mean @ Wl.T + bl + x_dst @ Wr.T


def _hetero(x, edges, Wl, bl, Wr):
    out = {t: None for t in x}
    for i, (s, d, name) in enumerate(_EDGE_TYPES):
        src, dst = edges[name]
        y = _sage(x[s], x[d], src, dst, Wl[i], bl[i], Wr[i])
        out[d] = y if out[d] is None else out[d] + y
    return out


def kernel(x_track, x_artist, x_genre, ei_made_by, ei_rev_made_by, ei_tagged, ei_rev_tagged, ei_plays, ei_rev_plays, ei_co_genre, proj_W, proj_b, conv1_Wl, conv1_bl, conv1_Wr, conv2_Wl, conv2_bl, conv2_Wr, norm1_g, norm1_b, norm2_g, norm2_b, head_W1, head_b1, head_W2, head_b2):
    raw = {'ei_made_by': ei_made_by, 'ei_rev_made_by': ei_rev_made_by, 'ei_tagged': ei_tagged, 'ei_rev_tagged': ei_rev_tagged, 'ei_plays': ei_plays, 'ei_rev_plays': ei_rev_plays, 'ei_co_genre': ei_co_genre}
    edges = {}
    for name, ei in raw.items():
        src = ei[0].astype(jnp.int32)
        dst = ei[1].astype(jnp.int32)
        dst_s, src_s = jax.lax.sort([dst, src], num_keys=1)
        edges[name] = (src_s, dst_s)

    x = {'track': jax.nn.relu(x_track @ proj_W[0].T + proj_b[0]),
         'artist': jax.nn.relu(x_artist @ proj_W[1].T + proj_b[1]),
         'genre': jax.nn.relu(x_genre @ proj_W[2].T + proj_b[2])}
    h = _hetero(x, edges, conv1_Wl, conv1_bl, conv1_Wr)
    h = {t: _layernorm(jax.nn.relu(h[t]), norm1_g[i], norm1_b[i]) for i, t in enumerate(_NTYPES)}
    h = _hetero(h, edges, conv2_Wl, conv2_bl, conv2_Wr)
    h = {t: _layernorm(jax.nn.relu(h[t]), norm2_g[i], norm2_b[i]) for i, t in enumerate(_NTYPES)}
    track = jax.nn.relu(h['track'] @ head_W1.T + head_b1) @ head_W2.T + head_b2
    return track, h['artist'], h['genre']


# speedup vs baseline: 2.7185x; 2.5283x over previous
import functools

import jax
import jax.numpy as jnp
from jax import lax
from jax.experimental import pallas as pl
from jax.experimental.pallas import tpu as pltpu
from jax.experimental.pallas import tpu_sc as plsc

D = 128
NT, NA, NG = 100000, 10000, 1000
CH = 4096
CHA = CH + 128
F32 = jnp.float32
I32 = jnp.int32

_SPECS = (
    ('mb', 0, 3, True),
    ('rmb', 1, 25, False),
    ('tg', 0, 1, True),
    ('rtg', 2, 25, False),
    ('pl', 1, 1, True),
    ('rp', 2, 3, True),
    ('cog', 0, 25, False),
)
_ROWS_ACC_PT = CHA // 16
_ROWS_OUT_PT = CH // 16


def _rup(x, m):
    return (x + m - 1) // m * m


@functools.cache
def _sc_layer(e_pads, with_cnt):
    mesh = plsc.VectorSubcoreMesh(core_axis_name="c", subcore_axis_name="s")
    out_type = []
    for (_, _, nch, partial) in _SPECS:
        shp = (2, nch * CH, D) if partial else (nch * CH, D)
        out_type.append(jax.ShapeDtypeStruct(shp, F32))
    if with_cnt:
        for (_, _, nch, partial) in _SPECS:
            shp = (2 * nch * CH,) if partial else (nch * CH,)
            out_type.append(jax.ShapeDtypeStruct(shp, F32))
    scratch = [
        pltpu.VMEM((32, 16), I32),
        pltpu.VMEM((512,), I32),
        pltpu.VMEM((512,), I32),
        pltpu.VMEM((512,), I32),
        pltpu.VMEM((512, D), F32),
        pltpu.VMEM((512,), F32),
        pltpu.VMEM((64, D), F32),
        pltpu.VMEM((64, D), F32),
        pltpu.VMEM((512,), F32),
        pltpu.VMEM((512,), F32),
        pltpu.VMEM_SHARED((CHA, D), F32),
        pltpu.VMEM_SHARED((CHA,), F32),
        pltpu.SemaphoreType.DMA,
    ]

    def body(xt, xa, xg, *rest):
        edges = rest[:21]
        rest = rest[21:]
        outs = rest[:7]
        rest = rest[7:]
        if with_cnt:
            couts = rest[:7]
            rest = rest[7:]
        else:
            couts = (None,) * 7
        (offs2_v, src_v, dst_v, rel_v, rows_v, ones_v, zero_v, stage_v,
         czero_v, cstage_v, acc_s, cnt_s, sem) = rest
        tables = (xt, xa, xg)
        c = lax.axis_index("c")
        s = lax.axis_index("s")
        one16 = jnp.ones((16,), F32)
        z16 = jnp.zeros((16,), F32)
        for i in range(32):
            ones_v[pl.ds(i * 16, 16)] = one16
            czero_v[pl.ds(i * 16, 16)] = z16
        for r in range(64):
            for j in range(8):
                zero_v[r, pl.ds(j * 16, 16)] = z16
        iota = lax.iota(I32, 16)

        def run_chunk(src_h, dst_h, table_h, out_h, cnt_h, partial, cnum,
                      cstride):
            vv = offs2_v[cnum]
            start = vv[0]
            end = vv[1]
            base = cnum * CH
            r0a = s * _ROWS_ACC_PT

            def z_(kk, c2):
                pltpu.sync_copy(zero_v, acc_s.at[pl.ds(r0a + kk * 64, 64)])
                return c2
            lax.fori_loop(0, _ROWS_ACC_PT // 64, z_, 0)
            rz = _ROWS_ACC_PT % 64
            if rz:
                pltpu.sync_copy(zero_v.at[pl.ds(0, rz)],
                                acc_s.at[pl.ds(r0a + _ROWS_ACC_PT - rz, rz)])
            if cnt_h is not None:
                pltpu.sync_copy(czero_v.at[pl.ds(0, _ROWS_ACC_PT)],
                                cnt_s.at[pl.ds(r0a, _ROWS_ACC_PT)])
            plsc.subcore_barrier()

            nworkers = 32 if partial else 16
            w = s * 2 + c if partial else s
            share = (end - start + nworkers - 1) // nworkers
            my_lo = start + w * share
            my_hi = jnp.minimum(my_lo + share, end)
            dma_lo = (my_lo // 8) * 8
            span = jnp.maximum(my_hi - dma_lo, 0)
            trips = (span + 511) // 512

            def blk(i, carry):
                off = dma_lo + i * 512
                pltpu.sync_copy(src_h.at[pl.ds(off, 512)], src_v)
                pltpu.sync_copy(dst_h.at[pl.ds(off, 512)], dst_v)

                def grp(g, c2):
                    o16 = g * 16
                    d16 = dst_v[pl.ds(o16, 16)]
                    pos = off + o16 + iota
                    m = (pos >= my_lo) & (pos < my_hi)
                    rel_v[pl.ds(o16, 16)] = jnp.where(m, d16 - base, CH + iota)
                    return c2

                lax.fori_loop(0, 32, grp, 0)
                pltpu.async_copy(table_h.at[src_v], rows_v, sem).wait()
                pltpu.sync_copy(rows_v, acc_s.at[rel_v], add=True)
                if cnt_h is not None:
                    pltpu.sync_copy(ones_v, cnt_s.at[rel_v], add=True)
                return carry

            lax.fori_loop(0, trips, blk, 0)
            plsc.subcore_barrier()

            r0o = s * _ROWS_OUT_PT

            def co_(kk, c2):
                rr = r0o + kk * 64
                pltpu.sync_copy(acc_s.at[pl.ds(rr, 64)], stage_v)
                if partial:
                    pltpu.sync_copy(stage_v, out_h.at[c, pl.ds(base + rr, 64)])
                else:
                    pltpu.sync_copy(stage_v, out_h.at[pl.ds(base + rr, 64)])
                return c2
            lax.fori_loop(0, _ROWS_OUT_PT // 64, co_, 0)
            ro = _ROWS_OUT_PT % 64
            if ro:
                rr = r0o + _ROWS_OUT_PT - ro
                pltpu.sync_copy(acc_s.at[pl.ds(rr, ro)], stage_v.at[pl.ds(0, ro)])
                if partial:
                    pltpu.sync_copy(stage_v.at[pl.ds(0, ro)],
                                    out_h.at[c, pl.ds(base + rr, ro)])
                else:
                    pltpu.sync_copy(stage_v.at[pl.ds(0, ro)],
                                    out_h.at[pl.ds(base + rr, ro)])
            if cnt_h is not None:
                pltpu.sync_copy(cnt_s.at[pl.ds(r0o, _ROWS_OUT_PT)],
                                cstage_v.at[pl.ds(0, _ROWS_OUT_PT)])
                cb = c * cstride + base + r0o if partial else base + r0o
                pltpu.sync_copy(cstage_v.at[pl.ds(0, _ROWS_OUT_PT)],
                                cnt_h.at[pl.ds(cb, _ROWS_OUT_PT)])
            plsc.subcore_barrier()

        for t, (key, tbl, nch, partial) in enumerate(_SPECS):
            src_h, dst_h, offs_h = edges[3 * t], edges[3 * t + 1], edges[3 * t + 2]
            pltpu.sync_copy(offs_h, offs2_v)
            if partial:
                def ploop(cn, c2, t=t, tbl=tbl, nch=nch,
                          src_h=src_h, dst_h=dst_h):
                    run_chunk(src_h, dst_h, tables[tbl], outs[t], couts[t],
                              True, cn, nch * CH)
                    return c2
                lax.fori_loop(0, nch, ploop, 0)
            else:
                def qloop(k, c2, t=t, tbl=tbl, nch=nch,
                          src_h=src_h, dst_h=dst_h):
                    cnum = c + 2 * k

                    @pl.when(cnum < nch)
                    def _():
                        run_chunk(src_h, dst_h, tables[tbl], outs[t], couts[t],
                                  False, cnum, 0)
                    return c2
                lax.fori_loop(0, (nch + 1) // 2, qloop, 0)

    return pl.kernel(body, out_type=tuple(out_type), mesh=mesh,
                     scratch_types=scratch)


def _tc_proj(x, W, b, ti):
    n = x.shape[0]
    R = 512
    grid = -(-n // R)

    def body(x_r, w_r, b_r, o_r):
        y = lax.dot_general(x_r[...], w_r[ti], (((1,), (1,)), ((), ())),
                            preferred_element_type=F32)
        o_r[...] = jnp.maximum(y + b_r[ti][None, :], 0.0)

    return pl.pallas_call(
        body,
        grid=(grid,),
        in_specs=[pl.BlockSpec((R, D), lambda i: (i, 0)),
                  pl.BlockSpec((3, D, D), lambda i: (0, 0, 0)),
                  pl.BlockSpec((3, D), lambda i: (0, 0))],
        out_specs=pl.BlockSpec((R, D), lambda i: (i, 0)),
        out_shape=jax.ShapeDtypeStruct((n, D), F32),
    )(x, W, b)


def _layernorm_rows(y, g_row, b_row):
    mu = jnp.mean(y, axis=-1, keepdims=True)
    var = jnp.mean((y - mu) ** 2, axis=-1, keepdims=True)
    return (y - mu) * lax.rsqrt(var + 1e-5) * g_row[None, :] + b_row[None, :]


def _tc_combine_dual(x, aggs, cnts, Wl, bl, Wr, ng, nb, idxs, ti):
    n = x.shape[0]
    R = 256
    grid = -(-n // R)
    i1, i2 = idxs

    def body(x_r, a1, c1, a2, c2, wl_r, bl_r, wr_r, g_r, bn_r, o_r):
        wr = wr_r[i1] + wr_r[i2]
        acc = lax.dot_general(x_r[...], wr, (((1,), (1,)), ((), ())),
                              preferred_element_type=F32)
        acc += (bl_r[i1] + bl_r[i2])[None, :]
        for a_r, c_r, ii in ((a1, c1, i1), (a2, c2, i2)):
            agg = a_r[0] + a_r[1]
            cnt = c_r[0] + c_r[1]
            mean = agg * (1.0 / jnp.maximum(cnt, 1.0))
            acc += lax.dot_general(mean, wl_r[ii], (((1,), (1,)), ((), ())),
                                   preferred_element_type=F32)
        o_r[...] = _layernorm_rows(jnp.maximum(acc, 0.0), g_r[ti], bn_r[ti])

    in_specs = [pl.BlockSpec((R, D), lambda i: (i, 0))]
    for _ in range(2):
        in_specs += [pl.BlockSpec((2, R, D), lambda i: (0, i, 0)),
                     pl.BlockSpec((2, R, 1), lambda i: (0, i, 0))]
    in_specs += [pl.BlockSpec((7, D, D), lambda i: (0, 0, 0)),
                 pl.BlockSpec((7, D), lambda i: (0, 0)),
                 pl.BlockSpec((7, D, D), lambda i: (0, 0, 0)),
                 pl.BlockSpec((3, D), lambda i: (0, 0)),
                 pl.BlockSpec((3, D), lambda i: (0, 0))]
    return pl.pallas_call(
        body, grid=(grid,), in_specs=in_specs,
        out_specs=pl.BlockSpec((R, D), lambda i: (i, 0)),
        out_shape=jax.ShapeDtypeStruct((n, D), F32),
    )(x, aggs[0], cnts[0], aggs[1], cnts[1], Wl, bl, Wr, ng, nb)


def _tc_combine_track(x, aggs, cnts, Wl, bl, Wr, ng, nb, idxs, ti):
    n = x.shape[0]
    R = 256
    grid = -(-n // R)
    i1, i2, i3 = idxs

    def body(x_r, a1, c1, a2, c2, a3, c3, wl_r, bl_r, wr_r, g_r, bn_r, o_r):
        wr = wr_r[i1] + wr_r[i2] + wr_r[i3]
        acc = lax.dot_general(x_r[...], wr, (((1,), (1,)), ((), ())),
                              preferred_element_type=F32)
        acc += (bl_r[i1] + bl_r[i2] + bl_r[i3])[None, :]
        for a_r, c_r, ii in ((a1, c1, i1), (a2, c2, i2), (a3, c3, i3)):
            mean = a_r[...] * (1.0 / jnp.maximum(c_r[...], 1.0))
            acc += lax.dot_general(mean, wl_r[ii], (((1,), (1,)), ((), ())),
                                   preferred_element_type=F32)
        o_r[...] = _layernorm_rows(jnp.maximum(acc, 0.0), g_r[ti], bn_r[ti])

    in_specs = [pl.BlockSpec((R, D), lambda i: (i, 0))]
    for _ in range(3):
        in_specs += [pl.BlockSpec((R, D), lambda i: (i, 0)),
                     pl.BlockSpec((R, 1), lambda i: (i, 0))]
    in_specs += [pl.BlockSpec((7, D, D), lambda i: (0, 0, 0)),
                 pl.BlockSpec((7, D), lambda i: (0, 0)),
                 pl.BlockSpec((7, D, D), lambda i: (0, 0, 0)),
                 pl.BlockSpec((3, D), lambda i: (0, 0)),
                 pl.BlockSpec((3, D), lambda i: (0, 0))]
    return pl.pallas_call(
        body, grid=(grid,), in_specs=in_specs,
        out_specs=pl.BlockSpec((R, D), lambda i: (i, 0)),
        out_shape=jax.ShapeDtypeStruct((n, D), F32),
    )(x, aggs[0], cnts[0], aggs[1], cnts[1], aggs[2], cnts[2],
      Wl, bl, Wr, ng, nb)


def _tc_head(x, W1, b1, W2, b2):
    n = x.shape[0]
    R = 512
    grid = -(-n // R)

    def body(x_r, w1_r, b1_r, w2_r, b2_r, o_r):
        t = lax.dot_general(x_r[...], w1_r[...], (((1,), (1,)), ((), ())),
                            preferred_element_type=F32)
        t = jnp.maximum(t + b1_r[...], 0.0)
        o_r[...] = lax.dot_general(t, w2_r[...], (((1,), (1,)), ((), ())),
                                   preferred_element_type=F32) + b2_r[...]

    return pl.pallas_call(
        body,
        grid=(grid,),
        in_specs=[pl.BlockSpec((R, D), lambda i: (i, 0)),
                  pl.BlockSpec((D, D), lambda i: (0, 0)),
                  pl.BlockSpec((1, D), lambda i: (0, 0)),
                  pl.BlockSpec((D, D), lambda i: (0, 0)),
                  pl.BlockSpec((1, D), lambda i: (0, 0))],
        out_specs=pl.BlockSpec((R, D), lambda i: (i, 0)),
        out_shape=jax.ShapeDtypeStruct((n, D), F32),
    )(x, W1, b1, W2, b2)


def _prep(e, n_chunks):
    src = e[0].astype(I32)
    dst = e[1].astype(I32)
    E = src.shape[0]
    dst_s, src_s = lax.sort([dst, src], num_keys=1)
    offs = jnp.searchsorted(dst_s, jnp.arange(n_chunks + 1, dtype=I32) * CH)
    offs = offs.astype(I32)
    offs2 = jnp.full((32, 16), E, I32)
    offs2 = offs2.at[:n_chunks, 0].set(offs[:-1]).at[:n_chunks, 1].set(offs[1:])
    e_pad = _rup(E + 512, 16)
    pad = e_pad - E
    psrc = jnp.arange(pad, dtype=I32) % 64
    pdst = jnp.full((pad,), n_chunks * CH, I32)
    return (jnp.concatenate([src_s, psrc]), jnp.concatenate([dst_s, pdst]),
            offs2, e_pad)


def kernel(x_track, x_artist, x_genre, ei_made_by, ei_rev_made_by, ei_tagged, ei_rev_tagged, ei_plays, ei_rev_plays, ei_co_genre, proj_W, proj_b, conv1_Wl, conv1_bl, conv1_Wr, conv2_Wl, conv2_bl, conv2_Wr, norm1_g, norm1_b, norm2_g, norm2_b, head_W1, head_b1, head_W2, head_b2):
    raw = (ei_made_by, ei_rev_made_by, ei_tagged, ei_rev_tagged, ei_plays,
           ei_rev_plays, ei_co_genre)
    prepped = [_prep(e, spec[2]) for e, spec in zip(raw, _SPECS)]
    e_pads = tuple(p[3] for p in prepped)
    edge_args = []
    for p in prepped:
        edge_args += [p[0], p[1], p[2]]

    xt = _tc_proj(x_track, proj_W, proj_b, 0)
    xa = _tc_proj(x_artist, proj_W, proj_b, 1)
    xg = _tc_proj(x_genre, proj_W, proj_b, 2)

    def layer(xs, Wl, bl, Wr, ng, nb, cnts):
        xt_, xa_, xg_ = xs
        first = cnts is None
        res = _sc_layer(e_pads, first)(xt_, xa_, xg_, *edge_args)
        aggs = dict(zip([sp[0] for sp in _SPECS], res[:7]))
        if first:
            cnts = {}
            for sp, a in zip(_SPECS, res[7:]):
                if sp[3]:
                    a = a.reshape(2, sp[2] * CH)
                cnts[sp[0]] = a[..., None]
        h_a = _tc_combine_dual(xa_, [aggs['mb'], aggs['rp']],
                               [cnts['mb'], cnts['rp']],
                               Wl, bl, Wr, ng, nb, (0, 5), 1)
        h_g = _tc_combine_dual(xg_, [aggs['tg'], aggs['pl']],
                               [cnts['tg'], cnts['pl']],
                               Wl, bl, Wr, ng, nb, (2, 4), 2)
        h_t = _tc_combine_track(xt_, [aggs['rmb'], aggs['rtg'], aggs['cog']],
                                [cnts['rmb'], cnts['rtg'], cnts['cog']],
                                Wl, bl, Wr, ng, nb, (1, 3, 6), 0)
        return (h_t, h_a, h_g), cnts

    h1, cnts = layer((xt, xa, xg), conv1_Wl, conv1_bl, conv1_Wr,
                     norm1_g, norm1_b, None)
    h2, _ = layer(h1, conv2_Wl, conv2_bl, conv2_Wr, norm2_g, norm2_b, cnts)
    track = _tc_head(h2[0], head_W1, head_b1.reshape(1, D),
                     head_W2, head_b2.reshape(1, D))
    return track, h2[1], h2[2]
